# baseline (speedup 1.0000x reference)
import math

import jax
import jax.numpy as jnp
from jax import lax
from jax.experimental import pallas as pl

_H = 4
_C = 32
_HC = _H * _C
_N = 10000
_E = 320000
_G = 64
_EB = 128
_INV_SQRT_C = 1.0 / math.sqrt(_C)


def _proj_body(h_ref, wq_ref, bq_ref, wk_ref, bk_ref, wv_ref, bv_ref,
               ws_ref, bs_ref, qs_ref, kt_ref, vt_ref, skip_ref):
  h = h_ref[...]
  qs_ref[...] = (jnp.dot(h, wq_ref[...], preferred_element_type=jnp.float32)
                 + bq_ref[...]) * _INV_SQRT_C
  kt_ref[...] = jnp.dot(h, wk_ref[...], preferred_element_type=jnp.float32) + bk_ref[...]
  vt_ref[...] = jnp.dot(h, wv_ref[...], preferred_element_type=jnp.float32) + bv_ref[...]
  skip_ref[...] = jnp.dot(h, ws_ref[...], preferred_element_type=jnp.float32) + bs_ref[...]


def _proj(h, wq, bq, wk, bk, wv, bv, ws, bs):
  rows = 2000
  grid = _N // rows
  full = lambda i: (0, 0)
  return pl.pallas_call(
      _proj_body,
      grid=(grid,),
      in_specs=[
          pl.BlockSpec((rows, _HC), lambda i: (i, 0)),
          pl.BlockSpec((_HC, _HC), full), pl.BlockSpec((1, _HC), full),
          pl.BlockSpec((_HC, _HC), full), pl.BlockSpec((1, _HC), full),
          pl.BlockSpec((_HC, _HC), full), pl.BlockSpec((1, _HC), full),
          pl.BlockSpec((_HC, _HC), full), pl.BlockSpec((1, _HC), full),
      ],
      out_specs=[
          pl.BlockSpec((rows, _HC), lambda i: (i, 0)),
          pl.BlockSpec((rows, _HC), lambda i: (i, 0)),
          pl.BlockSpec((rows, _HC), lambda i: (i, 0)),
          pl.BlockSpec((rows, _HC), lambda i: (i, 0)),
      ],
      out_shape=[
          jax.ShapeDtypeStruct((_N, _HC), jnp.float32),
          jax.ShapeDtypeStruct((_N, _HC), jnp.float32),
          jax.ShapeDtypeStruct((_N, _HC), jnp.float32),
          jax.ShapeDtypeStruct((_N, _HC), jnp.float32),
      ],
  )(h, wq, bq, wk, bk, wv, bv, ws, bs)


def _edge_tc_body(dst_ref, src_ref, ew_ref, qs_ref, kt_ref, vt_ref, wer_ref,
                  agg_ref, den_ref):
  i = pl.program_id(0)

  @pl.when(i == 0)
  def _():
    agg_ref[...] = jnp.zeros_like(agg_ref)
    den_ref[...] = jnp.zeros_like(den_ref)

  niota = lax.broadcasted_iota(jnp.int32, (_EB, _N), 1)
  oh_d = (dst_ref[...] == niota).astype(jnp.float32)
  oh_s = (src_ref[...] == niota).astype(jnp.float32)
  ew = ew_ref[...]
  e_emb = ew * wer_ref[...]
  hi = lax.Precision.HIGHEST
  q = jnp.dot(oh_d, qs_ref[...], preferred_element_type=jnp.float32, precision=hi)
  k = jnp.dot(oh_s, kt_ref[...], preferred_element_type=jnp.float32, precision=hi) + e_emb
  v = jnp.dot(oh_s, vt_ref[...], preferred_element_type=jnp.float32, precision=hi) + e_emb
  em = (lax.broadcasted_iota(jnp.int32, (_HC, _HC), 0) // _C
        == lax.broadcasted_iota(jnp.int32, (_HC, _HC), 1) // _C
        ).astype(jnp.float32)
  alpha = jnp.dot(q * k, em, preferred_element_type=jnp.float32,
                  precision=hi)
  exb = jnp.exp(alpha)
  msg = v * exb
  agg_ref[...] += lax.dot_general(oh_d, msg, (((0,), (0,)), ((), ())),
                                  preferred_element_type=jnp.float32,
                                  precision=hi)
  den_ref[...] += lax.dot_general(oh_d, exb, (((0,), (0,)), ((), ())),
                                  preferred_element_type=jnp.float32,
                                  precision=hi)


def _edge_tc(dst1, src1, ew1, qs, kt, vt, wer):
  grid = _E // _EB
  full = lambda i: (0, 0)
  return pl.pallas_call(
      _edge_tc_body,
      grid=(grid,),
      in_specs=[
          pl.BlockSpec((_EB, 1), lambda i: (i, 0)),
          pl.BlockSpec((_EB, 1), lambda i: (i, 0)),
          pl.BlockSpec((_EB, 1), lambda i: (i, 0)),
          pl.BlockSpec((_N, _HC), full),
          pl.BlockSpec((_N, _HC), full),
          pl.BlockSpec((_N, _HC), full),
          pl.BlockSpec((1, _HC), full),
      ],
      out_specs=[
          pl.BlockSpec((_N, _HC), full),
          pl.BlockSpec((_N, _HC), full),
      ],
      out_shape=[
          jax.ShapeDtypeStruct((_N, _HC), jnp.float32),
          jax.ShapeDtypeStruct((_N, _HC), jnp.float32),
      ],
  )(dst1, src1, ew1, qs, kt, vt, wer)


def _combine_bn(agg_ref, den_ref, skip_ref, gamma_ref, beta_ref):
  pre = agg_ref[...] / (den_ref[...] + 1e-16) + skip_ref[...]
  mu = jnp.mean(pre, axis=0, keepdims=True)
  var = jnp.mean((pre - mu) ** 2, axis=0, keepdims=True)
  return jnp.maximum(
      (pre - mu) / jnp.sqrt(var + 1e-5) * gamma_ref[...] + beta_ref[...], 0.0)


def _fin_body(agg_ref, den_ref, skip_ref, gamma_ref, beta_ref, out_ref):
  out_ref[...] = _combine_bn(agg_ref, den_ref, skip_ref, gamma_ref, beta_ref)


def _finalize(agg, den, skip, gamma, beta):
  return pl.pallas_call(
      _fin_body,
      out_shape=jax.ShapeDtypeStruct((_N, _HC), jnp.float32),
  )(agg, den, skip, gamma, beta)


def _head_body(agg_ref, den_ref, skip_ref, gamma_ref, beta_ref, batch_ref,
               wf1_ref, bf1_ref, wf2_ref, bf2_ref, out_ref):
  h = _combine_bn(agg_ref, den_ref, skip_ref, gamma_ref, beta_ref)
  onehot = (batch_ref[...] == lax.broadcasted_iota(jnp.int32, (_N, _G), 1)
            ).astype(jnp.float32)
  pooled = lax.dot_general(onehot, h, (((0,), (0,)), ((), ())),
                           preferred_element_type=jnp.float32,
                           precision=lax.Precision.HIGHEST)
  cnt = lax.dot_general(onehot, jnp.ones((_N, 1), jnp.float32),
                        (((0,), (0,)), ((), ())),
                        preferred_element_type=jnp.float32)
  pooled = pooled / jnp.maximum(cnt, 1.0)
  z = jnp.maximum(jnp.dot(pooled, wf1_ref[...],
                          preferred_element_type=jnp.float32) + bf1_ref[...], 0.0)
  out_ref[...] = (jnp.dot(z, wf2_ref[...], preferred_element_type=jnp.float32)
                  + bf2_ref[...])


def _head(agg, den, skip, gamma, beta, batch2d, wf1, bf1, wf2, bf2):
  return pl.pallas_call(
      _head_body,
      out_shape=jax.ShapeDtypeStruct((_G, 1), jnp.float32),
  )(agg, den, skip, gamma, beta, batch2d, wf1, bf1, wf2, bf2)


def kernel(x, edge_weight, params, edge_index, batch):
  dst1 = edge_index[1].reshape(_E, 1)
  src1 = edge_index[0].reshape(_E, 1)
  batch2d = batch.reshape(_N, 1)

  h = x
  n_layers = len(params["layers"])
  for li, lp in enumerate(params["layers"]):
    r2 = lambda b: b.reshape(1, _HC)
    qs, kt, vt, skip = _proj(h, lp["Wq"], r2(lp["bq"]), lp["Wk"],
                             r2(lp["bk"] + lp["be"]), lp["Wv"],
                             r2(lp["bv"] + lp["be"]), lp["Ws"], r2(lp["bs"]))
    agg, den = _edge_tc(dst1, src1, edge_weight, qs, kt, vt, lp["We"])
    gamma, beta = r2(lp["gamma"]), r2(lp["beta"])
    if li + 1 < n_layers:
      h = _finalize(agg, den, skip, gamma, beta)
    else:
      out = _head(agg, den, skip, gamma, beta, batch2d,
                  params["Wf1"], params["bf1"].reshape(1, _HC),
                  params["Wf2"], params["bf2"].reshape(1, 1))
  return out
